# Initial kernel scaffold; baseline (speedup 1.0000x reference)
import jax
import jax.numpy as jnp
from jax.experimental import pallas as pl


def kernel(efeat, nfeat, edge_index, W1, b1, W2, b2, ln_g, ln_b):
    raise NotImplementedError("write your pallas kernel here")


# speedup vs baseline: 2.3466x; 2.3466x over previous
import functools

import jax
import jax.numpy as jnp
from jax.experimental import pallas as pl
from jax.experimental.pallas import tpu as pltpu
from jax.experimental.pallas import tpu_sc as plsc

_HIGH = jax.lax.Precision.HIGHEST


def _proj_body(nfeat_ref, w1s_ref, w1d_ref, ps_ref, pd_ref):
    x = nfeat_ref[...]
    ps_ref[...] = jax.lax.dot(x, w1s_ref[...], precision=_HIGH,
                              preferred_element_type=jnp.float32)
    pd_ref[...] = jax.lax.dot(x, w1d_ref[...], precision=_HIGH,
                              preferred_element_type=jnp.float32)


def _project_nodes(nfeat, w1s, w1d):
    n, d = nfeat.shape
    out = jax.ShapeDtypeStruct((n, w1s.shape[1]), jnp.float32)
    return pl.pallas_call(
        _proj_body,
        out_shape=(out, out),
    )(nfeat, w1s, w1d)


def _gather_pairs(ps, pd, src2d, dst2d, gather_window):
    e = src2d.shape[1]
    d = ps.shape[1]
    out = jax.ShapeDtypeStruct((e, d), jnp.float32)
    mesh = plsc.VectorSubcoreMesh(core_axis_name="core",
                                  subcore_axis_name="subcore")

    @pl.kernel(out_type=(out, out), mesh=mesh)
    def kern(ps_hbm, pd_hbm, src_hbm, dst_hbm, gs_hbm, gd_hbm):
        def body(src_vmem, dst_vmem, gs_vmem, gd_vmem):
            pltpu.sync_copy(ps_hbm.at[src_vmem.at[0]], gs_vmem)
            pltpu.sync_copy(pd_hbm.at[dst_vmem.at[0]], gd_vmem)

        pltpu.emit_pipeline(
            body,
            grid=(e // gather_window,),
            in_specs=[pl.BlockSpec((1, gather_window), lambda i: (0, i)),
                      pl.BlockSpec((1, gather_window), lambda i: (0, i))],
            out_specs=[pl.BlockSpec((gather_window, d), lambda i: (i, 0)),
                       pl.BlockSpec((gather_window, d), lambda i: (i, 0))],
            core_axis_name=("core", "subcore"),
            dimension_semantics=(pltpu.PARALLEL,),
        )(src_hbm, dst_hbm, gs_hbm, gd_hbm)

    return kern(ps, pd, src2d, dst2d)


def _mlp_body(efeat_ref, gs_ref, gd_ref, w1e_ref, b1_ref, w2_ref, b2_ref,
              lng_ref, lnb_ref, out_ref):
    x = efeat_ref[...]
    pre = jax.lax.dot(x, w1e_ref[...], precision=_HIGH,
                      preferred_element_type=jnp.float32)
    pre = pre + gs_ref[...] + gd_ref[...] + b1_ref[...]
    h = pre * jax.lax.logistic(pre)
    y = jax.lax.dot(h, w2_ref[...], precision=_HIGH,
                    preferred_element_type=jnp.float32) + b2_ref[...]
    mu = jnp.mean(y, axis=-1, keepdims=True)
    yc = y - mu
    var = jnp.mean(yc * yc, axis=-1, keepdims=True)
    out = yc * jax.lax.rsqrt(var + 1e-5) * lng_ref[...] + lnb_ref[...]
    out_ref[...] = out + x


def _edge_mlp(efeat, gs, gd, w1e, b1, w2, b2, ln_g, ln_b, block_e):
    e, d = efeat.shape
    dh = w1e.shape[1]
    grid = (e // block_e,)
    edge_spec = pl.BlockSpec((block_e, d), lambda i: (i, 0))
    hid_spec = pl.BlockSpec((block_e, dh), lambda i: (i, 0))
    w_spec = lambda r, c: pl.BlockSpec((r, c), lambda i: (0, 0))
    return pl.pallas_call(
        _mlp_body,
        grid=grid,
        in_specs=[edge_spec, hid_spec, hid_spec,
                  w_spec(d, dh), w_spec(1, dh),
                  w_spec(dh, w2.shape[1]), w_spec(1, w2.shape[1]),
                  w_spec(1, w2.shape[1]), w_spec(1, w2.shape[1])],
        out_specs=edge_spec,
        out_shape=jax.ShapeDtypeStruct((e, d), jnp.float32),
    )(efeat, gs, gd, w1e, b1.reshape(1, -1), w2, b2.reshape(1, -1),
      ln_g.reshape(1, -1), ln_b.reshape(1, -1))


@functools.partial(jax.jit, static_argnames=())
def kernel(efeat, nfeat, edge_index, W1, b1, W2, b2, ln_g, ln_b):
    e, d_edge = efeat.shape
    d_node = nfeat.shape[1]
    w1e = W1[:d_edge]
    w1s = W1[d_edge:d_edge + d_node]
    w1d = W1[d_edge + d_node:]

    ps, pd = _project_nodes(nfeat, w1s, w1d)

    idx = edge_index.astype(jnp.int32)
    src2d = idx[0].reshape(1, e)
    dst2d = idx[1].reshape(1, e)
    gs, gd = _gather_pairs(ps, pd, src2d, dst2d, gather_window=128)

    efeat_new = _edge_mlp(efeat, gs, gd, w1e, b1, W2, b2, ln_g, ln_b,
                          block_e=1600)
    return (efeat_new, nfeat)


# speedup vs baseline: 2.4101x; 1.0270x over previous
import functools

import jax
import jax.numpy as jnp
from jax.experimental import pallas as pl
from jax.experimental.pallas import tpu as pltpu
from jax.experimental.pallas import tpu_sc as plsc

_HIGH = jax.lax.Precision.HIGHEST


def _proj_body(nfeat_ref, w1s_ref, w1d_ref, ps_ref, pd_ref):
    x = nfeat_ref[...]
    ps_ref[...] = jax.lax.dot(x, w1s_ref[...], precision=_HIGH,
                              preferred_element_type=jnp.float32)
    pd_ref[...] = jax.lax.dot(x, w1d_ref[...], precision=_HIGH,
                              preferred_element_type=jnp.float32)


def _project_nodes(nfeat, w1s, w1d):
    n, d = nfeat.shape
    out = jax.ShapeDtypeStruct((n, w1s.shape[1]), jnp.float32)
    return pl.pallas_call(
        _proj_body,
        out_shape=(out, out),
    )(nfeat, w1s, w1d)


def _gather_pairs(ps, pd, src2d, dst2d, gather_window):
    e = src2d.shape[1]
    d = ps.shape[1]
    out = jax.ShapeDtypeStruct((e, d), ps.dtype)
    mesh = plsc.VectorSubcoreMesh(core_axis_name="core",
                                  subcore_axis_name="subcore")

    @pl.kernel(out_type=out, mesh=mesh)
    def kern(ps_hbm, pd_hbm, src_hbm, dst_hbm, g_hbm):
        def body(src_vmem, dst_vmem, g_vmem):
            pltpu.sync_copy(ps_hbm.at[src_vmem.at[0]], g_vmem)
            pltpu.sync_copy(pd_hbm.at[dst_vmem.at[0]], g_vmem, add=True)

        pltpu.emit_pipeline(
            body,
            grid=(e // gather_window,),
            in_specs=[pl.BlockSpec((1, gather_window), lambda i: (0, i)),
                      pl.BlockSpec((1, gather_window), lambda i: (0, i))],
            out_specs=[pl.BlockSpec((gather_window, d), lambda i: (i, 0))],
            core_axis_name=("core", "subcore"),
            dimension_semantics=(pltpu.PARALLEL,),
        )(src_hbm, dst_hbm, g_hbm)

    return kern(ps, pd, src2d, dst2d)


def _mlp_body(efeat_ref, g_ref, w1e_ref, b1_ref, w2_ref, b2_ref,
              lng_ref, lnb_ref, out_ref):
    x = efeat_ref[...]
    pre = jax.lax.dot(x, w1e_ref[...], precision=_HIGH,
                      preferred_element_type=jnp.float32)
    pre = pre + g_ref[...] + b1_ref[...]
    h = pre * jax.lax.logistic(pre)
    y = jax.lax.dot(h, w2_ref[...], precision=_HIGH,
                    preferred_element_type=jnp.float32) + b2_ref[...]
    mu = jnp.mean(y, axis=-1, keepdims=True)
    yc = y - mu
    var = jnp.mean(yc * yc, axis=-1, keepdims=True)
    out = yc * jax.lax.rsqrt(var + 1e-5) * lng_ref[...] + lnb_ref[...]
    out_ref[...] = out + x


def _edge_mlp(efeat, g, w1e, b1, w2, b2, ln_g, ln_b, block_e):
    e, d = efeat.shape
    dh = w1e.shape[1]
    grid = (e // block_e,)
    edge_spec = pl.BlockSpec((block_e, d), lambda i: (i, 0))
    hid_spec = pl.BlockSpec((block_e, dh), lambda i: (i, 0))
    w_spec = lambda r, c: pl.BlockSpec((r, c), lambda i: (0, 0))
    return pl.pallas_call(
        _mlp_body,
        grid=grid,
        in_specs=[edge_spec, hid_spec,
                  w_spec(d, dh), w_spec(1, dh),
                  w_spec(dh, w2.shape[1]), w_spec(1, w2.shape[1]),
                  w_spec(1, w2.shape[1]), w_spec(1, w2.shape[1])],
        out_specs=edge_spec,
        out_shape=jax.ShapeDtypeStruct((e, d), jnp.float32),
    )(efeat, g, w1e, b1.reshape(1, -1), w2, b2.reshape(1, -1),
      ln_g.reshape(1, -1), ln_b.reshape(1, -1))


@functools.partial(jax.jit, static_argnames=())
def kernel(efeat, nfeat, edge_index, W1, b1, W2, b2, ln_g, ln_b):
    e, d_edge = efeat.shape
    d_node = nfeat.shape[1]
    w1e = W1[:d_edge]
    w1s = W1[d_edge:d_edge + d_node]
    w1d = W1[d_edge + d_node:]

    ps, pd = _project_nodes(nfeat, w1s, w1d)

    idx = edge_index.astype(jnp.int32)
    src2d = idx[0].reshape(1, e)
    dst2d = idx[1].reshape(1, e)
    g = _gather_pairs(ps, pd, src2d, dst2d, gather_window=128)

    efeat_new = _edge_mlp(efeat, g, w1e, b1, W2, b2, ln_g, ln_b,
                          block_e=1600)
    return (efeat_new, nfeat)


# speedup vs baseline: 3.8220x; 1.5858x over previous
import functools

import jax
import jax.numpy as jnp
from jax.experimental import pallas as pl
from jax.experimental.pallas import tpu as pltpu
from jax.experimental.pallas import tpu_sc as plsc

_HIGH = jax.lax.Precision.HIGHEST


def _proj_body(nfeat_ref, w1s_ref, w1d_ref, ps_ref, pd_ref):
    x = nfeat_ref[...]
    ps_ref[...] = jax.lax.dot(x, w1s_ref[...], precision=_HIGH,
                              preferred_element_type=jnp.float32)
    pd_ref[...] = jax.lax.dot(x, w1d_ref[...], precision=_HIGH,
                              preferred_element_type=jnp.float32)


def _project_nodes(nfeat, w1s, w1d):
    n, d = nfeat.shape
    out = jax.ShapeDtypeStruct((n, w1s.shape[1]), jnp.float32)
    return pl.pallas_call(
        _proj_body,
        out_shape=(out, out),
    )(nfeat, w1s, w1d)


def _gather_pairs(ps, pd, src2d, dst2d, gather_window):
    e = src2d.shape[1]
    d = ps.shape[1]
    out = jax.ShapeDtypeStruct((e, d), ps.dtype)
    mesh = plsc.VectorSubcoreMesh(core_axis_name="core",
                                  subcore_axis_name="subcore")

    @pl.kernel(out_type=out, mesh=mesh)
    def kern(ps_hbm, pd_hbm, src_hbm, dst_hbm, g_hbm):
        def body(src_vmem, dst_vmem, g_vmem):
            pltpu.sync_copy(ps_hbm.at[src_vmem.at[0]], g_vmem)
            pltpu.sync_copy(pd_hbm.at[dst_vmem.at[0]], g_vmem, add=True)

        pltpu.emit_pipeline(
            body,
            grid=(e // gather_window,),
            in_specs=[pl.BlockSpec((1, gather_window), lambda i: (0, i)),
                      pl.BlockSpec((1, gather_window), lambda i: (0, i))],
            out_specs=[pl.BlockSpec((gather_window, d), lambda i: (i, 0))],
            core_axis_name=("core", "subcore"),
            dimension_semantics=(pltpu.PARALLEL,),
        )(src_hbm, dst_hbm, g_hbm)

    return kern(ps, pd, src2d, dst2d)


def _mlp_body(efeat_ref, g_ref, w1e_ref, b1_ref, w2_ref, b2_ref,
              lng_ref, lnb_ref, out_ref):
    x = efeat_ref[...]
    pre = jax.lax.dot(x, w1e_ref[...],
                      preferred_element_type=jnp.float32)
    pre = pre + g_ref[...] + b1_ref[...]
    h = pre * jax.lax.logistic(pre)
    y = jax.lax.dot(h, w2_ref[...],
                    preferred_element_type=jnp.float32) + b2_ref[...]
    mu = jnp.mean(y, axis=-1, keepdims=True)
    yc = y - mu
    var = jnp.mean(yc * yc, axis=-1, keepdims=True)
    out = yc * jax.lax.rsqrt(var + 1e-5) * lng_ref[...] + lnb_ref[...]
    out_ref[...] = out + x


def _edge_mlp(efeat, g, w1e, b1, w2, b2, ln_g, ln_b, block_e):
    e, d = efeat.shape
    dh = w1e.shape[1]
    grid = (e // block_e,)
    edge_spec = pl.BlockSpec((block_e, d), lambda i: (i, 0))
    hid_spec = pl.BlockSpec((block_e, dh), lambda i: (i, 0))
    w_spec = lambda r, c: pl.BlockSpec((r, c), lambda i: (0, 0))
    return pl.pallas_call(
        _mlp_body,
        grid=grid,
        in_specs=[edge_spec, hid_spec,
                  w_spec(d, dh), w_spec(1, dh),
                  w_spec(dh, w2.shape[1]), w_spec(1, w2.shape[1]),
                  w_spec(1, w2.shape[1]), w_spec(1, w2.shape[1])],
        out_specs=edge_spec,
        out_shape=jax.ShapeDtypeStruct((e, d), jnp.float32),
        compiler_params=pltpu.CompilerParams(
            dimension_semantics=("parallel",)),
    )(efeat, g, w1e, b1.reshape(1, -1), w2, b2.reshape(1, -1),
      ln_g.reshape(1, -1), ln_b.reshape(1, -1))


@functools.partial(jax.jit, static_argnames=())
def kernel(efeat, nfeat, edge_index, W1, b1, W2, b2, ln_g, ln_b):
    e, d_edge = efeat.shape
    d_node = nfeat.shape[1]
    w1e = W1[:d_edge]
    w1s = W1[d_edge:d_edge + d_node]
    w1d = W1[d_edge + d_node:]

    ps, pd = _project_nodes(nfeat, w1s, w1d)

    idx = edge_index.astype(jnp.int32)
    src2d = idx[0].reshape(1, e)
    dst2d = idx[1].reshape(1, e)
    g = _gather_pairs(ps, pd, src2d, dst2d, gather_window=128)

    efeat_new = _edge_mlp(efeat, g, w1e, b1, W2, b2, ln_g, ln_b,
                          block_e=1600)
    return (efeat_new, nfeat)


# speedup vs baseline: 4.8519x; 1.2695x over previous
import functools

import jax
import jax.numpy as jnp
from jax.experimental import pallas as pl
from jax.experimental.pallas import tpu as pltpu
from jax.experimental.pallas import tpu_sc as plsc

_HIGH = jax.lax.Precision.HIGHEST


def _proj_body(nfeat_ref, w1s_ref, w1d_ref, ps_ref, pd_ref):
    x = nfeat_ref[...]
    ps_ref[...] = jax.lax.dot(x, w1s_ref[...], precision=_HIGH,
                              preferred_element_type=jnp.float32)
    pd_ref[...] = jax.lax.dot(x, w1d_ref[...], precision=_HIGH,
                              preferred_element_type=jnp.float32)


def _project_nodes(nfeat, w1s, w1d):
    n, d = nfeat.shape
    out = jax.ShapeDtypeStruct((n, w1s.shape[1]), jnp.float32)
    return pl.pallas_call(
        _proj_body,
        out_shape=(out, out),
    )(nfeat, w1s, w1d)


def _gather_pairs(ps, pd, src2d, dst2d, gather_window):
    e = src2d.shape[1]
    d = ps.shape[1]
    out = jax.ShapeDtypeStruct((e, d), ps.dtype)
    mesh = plsc.VectorSubcoreMesh(core_axis_name="core",
                                  subcore_axis_name="subcore")

    @pl.kernel(out_type=out, mesh=mesh)
    def kern(ps_hbm, pd_hbm, src_hbm, dst_hbm, g_hbm):
        def body(src_vmem, dst_vmem, g_vmem):
            pltpu.sync_copy(ps_hbm.at[src_vmem.at[0]], g_vmem)
            pltpu.sync_copy(pd_hbm.at[dst_vmem.at[0]], g_vmem, add=True)

        pltpu.emit_pipeline(
            body,
            grid=(e // gather_window,),
            in_specs=[pl.BlockSpec((1, gather_window), lambda i: (0, i)),
                      pl.BlockSpec((1, gather_window), lambda i: (0, i))],
            out_specs=[pl.BlockSpec((gather_window, d), lambda i: (i, 0))],
            core_axis_name=("core", "subcore"),
            dimension_semantics=(pltpu.PARALLEL,),
        )(src_hbm, dst_hbm, g_hbm)

    return kern(ps, pd, src2d, dst2d)


def _mlp_body(efeat_ref, g_ref, w1e_ref, b1_ref, w2_ref, b2_ref,
              lng_ref, lnb_ref, out_ref):
    x = efeat_ref[...]
    pre = jax.lax.dot(x, w1e_ref[...],
                      preferred_element_type=jnp.float32)
    pre = pre + g_ref[...] + b1_ref[...]
    h = pre * jax.lax.logistic(pre)
    y = jax.lax.dot(h, w2_ref[...],
                    preferred_element_type=jnp.float32) + b2_ref[...]
    mu = jnp.mean(y, axis=-1, keepdims=True)
    yc = y - mu
    var = jnp.mean(yc * yc, axis=-1, keepdims=True)
    out = yc * jax.lax.rsqrt(var + 1e-5) * lng_ref[...] + lnb_ref[...]
    out_ref[...] = out + x


def _mlp_body_buf(buf_ref, efeat_ref, g_ref, w1e_ref, b1_ref, w2_ref,
                  b2_ref, lng_ref, lnb_ref, out_ref):
    del buf_ref
    _mlp_body(efeat_ref, g_ref, w1e_ref, b1_ref, w2_ref, b2_ref,
              lng_ref, lnb_ref, out_ref)


def _edge_mlp_chunk(buf, efeat, g, w1e, b1, w2, b2, ln_g, ln_b,
                    block_e, off_blocks):
    e, d = efeat.shape
    chunk = g.shape[0]
    dh = w1e.shape[1]
    grid = (chunk // block_e,)
    edge_spec = pl.BlockSpec((block_e, d),
                             lambda i, o=off_blocks: (i + o, 0))
    g_spec = pl.BlockSpec((block_e, dh), lambda i: (i, 0))
    w_spec = lambda r, c: pl.BlockSpec((r, c), lambda i: (0, 0))
    in_specs = [edge_spec, g_spec,
                w_spec(d, dh), w_spec(1, dh),
                w_spec(dh, w2.shape[1]), w_spec(1, w2.shape[1]),
                w_spec(1, w2.shape[1]), w_spec(1, w2.shape[1])]
    operands = [efeat, g, w1e, b1.reshape(1, -1), w2, b2.reshape(1, -1),
                ln_g.reshape(1, -1), ln_b.reshape(1, -1)]
    body = _mlp_body
    aliases = {}
    if buf is not None:
        in_specs = [pl.BlockSpec(memory_space=pltpu.MemorySpace.HBM)
                    ] + in_specs
        operands = [buf] + operands
        body = _mlp_body_buf
        aliases = {0: 0}
    return pl.pallas_call(
        body,
        grid=grid,
        in_specs=in_specs,
        out_specs=edge_spec,
        out_shape=jax.ShapeDtypeStruct((e, d), jnp.float32),
        input_output_aliases=aliases,
        compiler_params=pltpu.CompilerParams(
            dimension_semantics=("parallel",)),
    )(*operands)


@functools.partial(jax.jit, static_argnames=())
def kernel(efeat, nfeat, edge_index, W1, b1, W2, b2, ln_g, ln_b):
    e, d_edge = efeat.shape
    d_node = nfeat.shape[1]
    w1e = W1[:d_edge]
    w1s = W1[d_edge:d_edge + d_node]
    w1d = W1[d_edge + d_node:]

    ps, pd = _project_nodes(nfeat, w1s, w1d)

    idx = edge_index.astype(jnp.int32)

    n_chunks = 4
    block_e = 1600
    chunk = e // n_chunks
    steps = chunk // block_e
    buf = None
    for k in range(n_chunks):
        src2d = jax.lax.slice(idx, (0, k * chunk), (1, (k + 1) * chunk))
        dst2d = jax.lax.slice(idx, (1, k * chunk), (2, (k + 1) * chunk))
        g = _gather_pairs(ps, pd, src2d, dst2d, gather_window=128)
        buf = _edge_mlp_chunk(buf, efeat, g, w1e, b1, W2, b2, ln_g, ln_b,
                              block_e=block_e, off_blocks=k * steps)
    return (buf, nfeat)
